# Initial kernel scaffold; baseline (speedup 1.0000x reference)
import jax
import jax.numpy as jnp
from jax.experimental import pallas as pl


def kernel(codes, codebooks, scales):
    raise NotImplementedError("write your pallas kernel here")


# speedup vs baseline: 135.4977x; 135.4977x over previous
import functools

import jax
import jax.numpy as jnp
from jax import lax
from jax.experimental import pallas as pl
from jax.experimental.pallas import tpu as pltpu
from jax.experimental.pallas import tpu_sc as plsc


def _build_sc_call(num_out, in_features, num_cb, cb_size):
  info = plsc.get_sparse_core_info()
  nc, ns, L = info.num_cores, info.num_subcores, info.num_lanes
  nw = nc * ns
  rows_per_w = num_out // nw
  num_in_groups = in_features // 8
  n_blocks = num_in_groups // L
  mesh = plsc.VectorSubcoreMesh(core_axis_name="c", subcore_axis_name="s")

  @functools.partial(
      pl.kernel,
      mesh=mesh,
      out_type=jax.ShapeDtypeStruct((num_out, in_features), jnp.float32),
      compiler_params=pltpu.CompilerParams(needs_layout_passes=False),
      scratch_types=[
          pltpu.VMEM((4, num_cb * cb_size), jnp.int32),
          pltpu.VMEM((in_features,), jnp.int32),
          pltpu.VMEM((in_features,), jnp.float32),
          pltpu.VMEM((rows_per_w,), jnp.float32),
      ],
  )
  def k(codes_hbm, tbl_hbm, scales_hbm, out_hbm, tbl_v, codes_v, out_v,
        scales_v):
    wid = lax.axis_index("s") * nc + lax.axis_index("c")
    row0 = wid * rows_per_w
    pltpu.sync_copy(tbl_hbm, tbl_v)
    pltpu.sync_copy(scales_hbm.at[pl.ds(row0, rows_per_w)], scales_v)
    lane8 = lax.iota(jnp.int32, L) * 8
    jp_idx = [jnp.full((L,), jp, jnp.int32) for jp in range(4)]

    def row_body(r, carry):
      pltpu.sync_copy(codes_hbm.at[row0 + r], codes_v)
      scale = plsc.load_gather(scales_v, [jnp.full((L,), r, jnp.int32)])

      def blk_body(b, carry2):
        base = b * (L * num_cb)
        accs = [jnp.zeros((L,), jnp.float32) for _ in range(8)]
        for m in range(num_cb):
          cm = plsc.load_gather(codes_v, [lane8 + (base + m)])
          cmo = cm + m * cb_size
          for jp in range(4):
            g = plsc.load_gather(tbl_v, [jp_idx[jp], cmo])
            a, b2 = plsc.unpack(
                plsc.bitcast(g, jnp.bfloat16),
                format=plsc.PackFormat.INTERLEAVED)
            accs[2 * jp] = accs[2 * jp] + a
            accs[2 * jp + 1] = accs[2 * jp + 1] + b2
        for j in range(8):
          plsc.store_scatter(out_v, [lane8 + (base + j)], accs[j] * scale)
        return carry2

      lax.fori_loop(0, n_blocks, blk_body, 0, unroll=False)
      pltpu.sync_copy(out_v, out_hbm.at[row0 + r])
      return carry

    lax.fori_loop(0, rows_per_w, row_body, 0, unroll=False)

  return k


def kernel(codes, codebooks, scales):
  num_out, num_in_groups, num_cb = codes.shape
  _, cb_size, out_group, in_group = codebooks.shape
  in_features = num_in_groups * in_group

  codes_flat = codes.reshape(num_out, num_in_groups * num_cb)
  scales_flat = scales.reshape(num_out)
  cb = codebooks.reshape(num_cb, cb_size, in_group).astype(jnp.bfloat16)
  cb = cb.reshape(num_cb, cb_size, in_group // 2, 2).transpose(2, 0, 1, 3)
  tbl = lax.bitcast_convert_type(cb, jnp.int32)
  tbl = tbl.reshape(in_group // 2, num_cb * cb_size)

  call = _build_sc_call(num_out, in_features, num_cb, cb_size)
  return call(codes_flat, tbl, scales_flat)
